# baseline (speedup 1.0000x reference)
import numpy as np
import jax
import jax.numpy as jnp
from jax.experimental import pallas as pl
from jax.experimental.pallas import tpu as pltpu

_VMEM_BUDGET = int(64 * 1024 * 1024 * 0.7)


def _pick_batch(N, per_image_bytes, extra_bytes):
    for B in (8, 4, 2, 1):
        if N % B:
            continue
        if 4 * B * per_image_bytes + extra_bytes + (2 << 20) <= _VMEM_BUDGET:
            return B
    return 1


def _make_packed_kernel(B, P, inv_hw):
    def se_kernel(x_ref, w1_ref, b1_ref, w2_ref, b2_ref, s_ref, st_ref,
                  e_ref, et_ref, m_ref, o_ref):
        R = x_ref.shape[1]
        L = x_ref.shape[2]

        x_all = x_ref[...].reshape(B * R, L)
        y = jnp.dot(x_all, s_ref[...], preferred_element_type=jnp.float32)

        cols = []
        m = m_ref[...]
        for b in range(B):
            z = jnp.dot(e_ref[...], y[b * R:(b + 1) * R, :],
                        preferred_element_type=jnp.float32)
            cols.append(jnp.sum(z * m, axis=1, keepdims=True))
        pooled = jnp.concatenate(cols, axis=1) * inv_hw

        h = jnp.dot(w1_ref[...], pooled,
                    preferred_element_type=jnp.float32) + b1_ref[...]
        h = h * jax.nn.sigmoid(h)
        g = jnp.dot(w2_ref[...], h,
                    preferred_element_type=jnp.float32) + b2_ref[...]
        g = jax.nn.sigmoid(g)

        for b in range(B):
            gm = g[:, b:b + 1] * m
            gb = jnp.dot(et_ref[...], gm,
                         preferred_element_type=jnp.float32)
            gate = jnp.dot(gb, st_ref[...],
                           preferred_element_type=jnp.float32)
            o_ref[b] = x_ref[b] * gate.astype(o_ref.dtype)

    return se_kernel


def _packed_forward(x3, w1, b1, w2, b2, P):
    N, C, HW = x3.shape
    Cse = w1.shape[0]
    R = C // P
    L = P * HW
    itemsize = jnp.dtype(x3.dtype).itemsize

    xp = x3.reshape(N, R, L)
    w1f = w1.astype(jnp.float32)
    w2f = w2.astype(jnp.float32)
    b1c = b1.reshape(Cse, 1).astype(jnp.float32)
    b2c = b2.reshape(C, 1).astype(jnp.float32)

    seg = np.repeat(np.arange(P), HW)
    s_np = (seg[:, None] == np.arange(P)[None, :]).astype(np.float32)
    s = jnp.asarray(s_np)
    st = jnp.asarray(s_np.T.copy())
    e_np = (np.arange(C)[:, None] // P ==
            np.arange(R)[None, :]).astype(np.float32)
    e = jnp.asarray(e_np)
    et = jnp.asarray(e_np.T.copy())
    m_np = (np.arange(C)[:, None] % P ==
            np.arange(P)[None, :]).astype(np.float32)
    m = jnp.asarray(m_np)

    extra = (2 * C * Cse + C + Cse + 2 * L * P + 2 * C * R + C * P) * 4
    B = _pick_batch(N, C * HW * itemsize, extra)

    out = pl.pallas_call(
        _make_packed_kernel(B, P, 1.0 / float(HW)),
        out_shape=jax.ShapeDtypeStruct((N, R, L), x3.dtype),
        grid=(N // B,),
        in_specs=[
            pl.BlockSpec((B, R, L), lambda n: (n, 0, 0)),
            pl.BlockSpec((Cse, C), lambda n: (0, 0)),
            pl.BlockSpec((Cse, 1), lambda n: (0, 0)),
            pl.BlockSpec((C, Cse), lambda n: (0, 0)),
            pl.BlockSpec((C, 1), lambda n: (0, 0)),
            pl.BlockSpec((L, P), lambda n: (0, 0)),
            pl.BlockSpec((P, L), lambda n: (0, 0)),
            pl.BlockSpec((C, R), lambda n: (0, 0)),
            pl.BlockSpec((R, C), lambda n: (0, 0)),
            pl.BlockSpec((C, P), lambda n: (0, 0)),
        ],
        out_specs=pl.BlockSpec((B, R, L), lambda n: (n, 0, 0)),
        compiler_params=pltpu.CompilerParams(
            dimension_semantics=("parallel",),
            vmem_limit_bytes=_VMEM_BUDGET),
    )(xp, w1f, b1c, w2f, b2c, s, st, e, et, m)
    return out.reshape(N, C, HW)


def _make_plain_kernel(B, inv_hw):
    def se_kernel(x_ref, w1_ref, b1_ref, w2_ref, b2_ref, o_ref):
        C = x_ref.shape[1]
        HW = x_ref.shape[2]

        x_flat = x_ref[...].reshape(B * C, HW)
        ones = jnp.ones((HW, 1), dtype=x_flat.dtype)
        pooled_col = jnp.dot(x_flat, ones, preferred_element_type=jnp.float32)
        pooled = jnp.concatenate(
            [pooled_col[b * C:(b + 1) * C, :] for b in range(B)], axis=1)
        pooled = pooled * inv_hw

        h = jnp.dot(w1_ref[...], pooled,
                    preferred_element_type=jnp.float32) + b1_ref[...]
        h = h * jax.nn.sigmoid(h)
        g = jnp.dot(w2_ref[...], h,
                    preferred_element_type=jnp.float32) + b2_ref[...]
        g = jax.nn.sigmoid(g).astype(o_ref.dtype)

        for b in range(B):
            o_ref[b] = x_ref[b] * g[:, b:b + 1]

    return se_kernel


def _plain_forward(x3, w1, b1, w2, b2):
    N, C, HW = x3.shape
    Cse = w1.shape[0]
    itemsize = jnp.dtype(x3.dtype).itemsize

    w1f = w1.astype(jnp.float32)
    w2f = w2.astype(jnp.float32)
    b1c = b1.reshape(Cse, 1).astype(jnp.float32)
    b2c = b2.reshape(C, 1).astype(jnp.float32)

    B = _pick_batch(N, C * HW * itemsize, (2 * C * Cse + C + Cse) * 4)

    out = pl.pallas_call(
        _make_plain_kernel(B, 1.0 / float(HW)),
        out_shape=jax.ShapeDtypeStruct((N, C, HW), x3.dtype),
        grid=(N // B,),
        in_specs=[
            pl.BlockSpec((B, C, HW), lambda n: (n, 0, 0)),
            pl.BlockSpec((Cse, C), lambda n: (0, 0)),
            pl.BlockSpec((Cse, 1), lambda n: (0, 0)),
            pl.BlockSpec((C, Cse), lambda n: (0, 0)),
            pl.BlockSpec((C, 1), lambda n: (0, 0)),
        ],
        out_specs=pl.BlockSpec((B, C, HW), lambda n: (n, 0, 0)),
        compiler_params=pltpu.CompilerParams(
            dimension_semantics=("parallel",),
            vmem_limit_bytes=_VMEM_BUDGET),
    )(x3, w1f, b1c, w2f, b2c)
    return out


def kernel(x_nchw, w1, b1, w2, b2):
    N, C, H, W = x_nchw.shape
    HW = H * W
    x3 = x_nchw.reshape(N, C, HW)

    P = 128 // int(np.gcd(HW, 128))
    if C % P == 0 and P * HW >= 128:
        out = _packed_forward(x3, w1, b1, w2, b2, P)
    else:
        out = _plain_forward(x3, w1, b1, w2, b2)
    return out.reshape(N, C, H, W)


# speedup vs baseline: 9.2803x; 9.2803x over previous
import numpy as np
import jax
import jax.numpy as jnp
from jax.experimental import pallas as pl
from jax.experimental.pallas import tpu as pltpu

_VMEM_BUDGET = int(64 * 1024 * 1024 * 0.7)


def _largest_divisor_tile(total, unit_bytes, target_bytes):
    best = 1
    for t in range(1, total + 1):
        if total % t == 0 and t * unit_bytes <= target_bytes:
            best = t
    return best


def _make_pool_kernel(num_tiles, inv_hw):
    def pool_kernel(x_ref, w1t_ref, b1_ref, w2t_ref, b2_ref, g_ref):
        t = pl.program_id(1)

        part = jnp.sum(x_ref[...].astype(jnp.float32), axis=0)

        @pl.when(t == 0)
        def _init():
            g_ref[...] = part

        @pl.when(t > 0)
        def _acc():
            g_ref[...] += part

        @pl.when(t == num_tiles - 1)
        def _excite():
            pooled = g_ref[...] * inv_hw
            h = jnp.dot(pooled, w1t_ref[...],
                        preferred_element_type=jnp.float32) + b1_ref[...]
            h = h * jax.nn.sigmoid(h)
            g = jnp.dot(h, w2t_ref[...],
                        preferred_element_type=jnp.float32) + b2_ref[...]
            g_ref[...] = jax.nn.sigmoid(g)

    return pool_kernel


def _scale_kernel(x_ref, g_ref, o_ref):
    o_ref[...] = x_ref[...] * g_ref[...].astype(o_ref.dtype)[None]


def kernel(x_nchw, w1, b1, w2, b2):
    N, C, H, W = x_nchw.shape
    Cse = w1.shape[0]
    HW = H * W
    itemsize = jnp.dtype(x_nchw.dtype).itemsize

    xt = jnp.transpose(x_nchw, (2, 3, 0, 1)).reshape(HW, N, C)

    w1t = w1.T.astype(jnp.float32)
    w2t = w2.T.astype(jnp.float32)
    b1r = b1.reshape(1, Cse).astype(jnp.float32)
    b2r = b2.reshape(1, C).astype(jnp.float32)

    npar = 2 if N % 16 == 0 else 1
    Np = N // npar

    slab = Np * C * itemsize
    t1 = _largest_divisor_tile(HW, slab, 4 << 20)
    n1 = HW // t1
    t2 = _largest_divisor_tile(HW, N * C * itemsize, 4 << 20)
    n2 = HW // (t2 * npar) if (HW // t2) % npar == 0 else None
    if n2 is None:
        t2 = t1
        n2 = HW // (t2 * npar) if (HW // t2) % npar == 0 else HW // t2

    gate = pl.pallas_call(
        _make_pool_kernel(n1, 1.0 / float(HW)),
        out_shape=jax.ShapeDtypeStruct((N, C), jnp.float32),
        grid=(npar, n1),
        in_specs=[
            pl.BlockSpec((t1, Np, C), lambda p, t: (t, p, 0)),
            pl.BlockSpec((C, Cse), lambda p, t: (0, 0)),
            pl.BlockSpec((1, Cse), lambda p, t: (0, 0)),
            pl.BlockSpec((Cse, C), lambda p, t: (0, 0)),
            pl.BlockSpec((1, C), lambda p, t: (0, 0)),
        ],
        out_specs=pl.BlockSpec((Np, C), lambda p, t: (p, 0)),
        compiler_params=pltpu.CompilerParams(
            dimension_semantics=("parallel", "arbitrary"),
            vmem_limit_bytes=_VMEM_BUDGET),
    )(xt, w1t, b1r, w2t, b2r)

    nblk = HW // t2
    if nblk % npar == 0:
        n2 = nblk // npar
        sgrid = (npar, n2)
        x_spec = pl.BlockSpec((t2, N, C), lambda p, t: (p * n2 + t, 0, 0))
        g_spec = pl.BlockSpec((N, C), lambda p, t: (0, 0))
        o_spec = pl.BlockSpec((t2, N, C), lambda p, t: (p * n2 + t, 0, 0))
        sems = ("parallel", "parallel")
    else:
        sgrid = (nblk,)
        x_spec = pl.BlockSpec((t2, N, C), lambda t: (t, 0, 0))
        g_spec = pl.BlockSpec((N, C), lambda t: (0, 0))
        o_spec = pl.BlockSpec((t2, N, C), lambda t: (t, 0, 0))
        sems = ("parallel",)

    outt = pl.pallas_call(
        _scale_kernel,
        out_shape=jax.ShapeDtypeStruct((HW, N, C), x_nchw.dtype),
        grid=sgrid,
        in_specs=[x_spec, g_spec],
        out_specs=o_spec,
        compiler_params=pltpu.CompilerParams(
            dimension_semantics=sems,
            vmem_limit_bytes=_VMEM_BUDGET),
    )(xt, gate)

    return jnp.transpose(outt.reshape(H, W, N, C), (2, 3, 0, 1))


# speedup vs baseline: 12.2249x; 1.3173x over previous
import jax
import jax.numpy as jnp
from jax.experimental import pallas as pl
from jax.experimental.pallas import tpu as pltpu

_VMEM_BUDGET = int(64 * 1024 * 1024 * 0.7)


def _largest_divisor_tile(total, unit_bytes, target_bytes):
    best = 1
    for t in range(1, total + 1):
        if total % t == 0 and t * unit_bytes <= target_bytes:
            best = t
    return best


def _mlp_gate(pooled, w1_ref, b1_ref, w2_ref, b2_ref):
    h = jax.lax.dot_general(
        pooled, w1_ref[...].astype(jnp.float32),
        (((1,), (1,)), ((), ())),
        preferred_element_type=jnp.float32) + b1_ref[...]
    h = h * jax.nn.sigmoid(h)
    g = jax.lax.dot_general(
        h, w2_ref[...].astype(jnp.float32),
        (((1,), (1,)), ((), ())),
        preferred_element_type=jnp.float32) + b2_ref[...]
    return jax.nn.sigmoid(g)


def _make_resident_kernel(n_tiles, tile, inv_hw):
    def se_kernel(x_ref, w1_ref, b1_ref, w2_ref, b2_ref, o_ref,
                  xs_ref, acc_ref):
        t = pl.program_id(1)

        @pl.when(t < n_tiles)
        def _pool_phase():
            x = x_ref[...]
            part = jnp.sum(x.astype(jnp.float32), axis=0)

            @pl.when(t == 0)
            def _init():
                acc_ref[...] = part

            @pl.when(t > 0)
            def _acc():
                acc_ref[...] += part

            xs_ref[pl.ds(t * tile, tile)] = x

        @pl.when(t == n_tiles - 1)
        def _excite():
            acc_ref[...] = _mlp_gate(acc_ref[...] * inv_hw,
                                     w1_ref, b1_ref, w2_ref, b2_ref)

        @pl.when(t >= n_tiles)
        def _scale_phase():
            j = t - n_tiles
            g = acc_ref[...].astype(o_ref.dtype)
            o_ref[...] = xs_ref[pl.ds(j * tile, tile)] * g[None]

    return se_kernel


def _resident_forward(xt, w1, b1r, w2, b2r, npar, t1):
    HW, N, C = xt.shape
    Cse = w1.shape[0]
    Np = N // npar
    n1 = HW // t1

    out = pl.pallas_call(
        _make_resident_kernel(n1, t1, 1.0 / float(HW)),
        out_shape=jax.ShapeDtypeStruct((HW, N, C), xt.dtype),
        grid=(npar, 2 * n1),
        in_specs=[
            pl.BlockSpec((t1, Np, C),
                         lambda p, t: (jnp.minimum(t, n1 - 1), p, 0)),
            pl.BlockSpec((Cse, C), lambda p, t: (0, 0)),
            pl.BlockSpec((1, Cse), lambda p, t: (0, 0)),
            pl.BlockSpec((C, Cse), lambda p, t: (0, 0)),
            pl.BlockSpec((1, C), lambda p, t: (0, 0)),
        ],
        out_specs=pl.BlockSpec((t1, Np, C),
                               lambda p, t: (jnp.maximum(t - n1, 0), p, 0)),
        scratch_shapes=[
            pltpu.VMEM((HW, Np, C), xt.dtype),
            pltpu.VMEM((Np, C), jnp.float32),
        ],
        compiler_params=pltpu.CompilerParams(
            dimension_semantics=("parallel", "arbitrary"),
            vmem_limit_bytes=_VMEM_BUDGET),
    )(xt, w1, b1r, w2, b2r)
    return out


def _make_pool_kernel(num_tiles, inv_hw):
    def pool_kernel(x_ref, w1_ref, b1_ref, w2_ref, b2_ref, g_ref):
        t = pl.program_id(1)
        part = jnp.sum(x_ref[...].astype(jnp.float32), axis=0)

        @pl.when(t == 0)
        def _init():
            g_ref[...] = part

        @pl.when(t > 0)
        def _acc():
            g_ref[...] += part

        @pl.when(t == num_tiles - 1)
        def _excite():
            g_ref[...] = _mlp_gate(g_ref[...] * inv_hw,
                                   w1_ref, b1_ref, w2_ref, b2_ref)

    return pool_kernel


def _scale_kernel(x_ref, g_ref, o_ref):
    o_ref[...] = x_ref[...] * g_ref[...].astype(o_ref.dtype)[None]


def _two_pass_forward(xt, w1, b1r, w2, b2r, npar, t1):
    HW, N, C = xt.shape
    Cse = w1.shape[0]
    Np = N // npar
    n1 = HW // t1
    itemsize = jnp.dtype(xt.dtype).itemsize

    gate = pl.pallas_call(
        _make_pool_kernel(n1, 1.0 / float(HW)),
        out_shape=jax.ShapeDtypeStruct((N, C), jnp.float32),
        grid=(npar, n1),
        in_specs=[
            pl.BlockSpec((t1, Np, C), lambda p, t: (t, p, 0)),
            pl.BlockSpec((Cse, C), lambda p, t: (0, 0)),
            pl.BlockSpec((1, Cse), lambda p, t: (0, 0)),
            pl.BlockSpec((C, Cse), lambda p, t: (0, 0)),
            pl.BlockSpec((1, C), lambda p, t: (0, 0)),
        ],
        out_specs=pl.BlockSpec((Np, C), lambda p, t: (p, 0)),
        compiler_params=pltpu.CompilerParams(
            dimension_semantics=("parallel", "arbitrary"),
            vmem_limit_bytes=_VMEM_BUDGET),
    )(xt, w1, b1r, w2, b2r)

    t2 = _largest_divisor_tile(HW, N * C * itemsize, 4 << 20)
    nblk = HW // t2
    if nblk % npar == 0:
        n2 = nblk // npar
        sgrid = (npar, n2)
        x_spec = pl.BlockSpec((t2, N, C), lambda p, t: (p * n2 + t, 0, 0))
        g_spec = pl.BlockSpec((N, C), lambda p, t: (0, 0))
        o_spec = pl.BlockSpec((t2, N, C), lambda p, t: (p * n2 + t, 0, 0))
        sems = ("parallel", "parallel")
    else:
        sgrid = (nblk,)
        x_spec = pl.BlockSpec((t2, N, C), lambda t: (t, 0, 0))
        g_spec = pl.BlockSpec((N, C), lambda t: (0, 0))
        o_spec = pl.BlockSpec((t2, N, C), lambda t: (t, 0, 0))
        sems = ("parallel",)

    outt = pl.pallas_call(
        _scale_kernel,
        out_shape=jax.ShapeDtypeStruct((HW, N, C), xt.dtype),
        grid=sgrid,
        in_specs=[x_spec, g_spec],
        out_specs=o_spec,
        compiler_params=pltpu.CompilerParams(
            dimension_semantics=sems,
            vmem_limit_bytes=_VMEM_BUDGET),
    )(xt, gate)
    return outt


def kernel(x_nchw, w1, b1, w2, b2):
    N, C, H, W = x_nchw.shape
    Cse = w1.shape[0]
    HW = H * W
    itemsize = jnp.dtype(x_nchw.dtype).itemsize

    xt = jnp.transpose(x_nchw, (2, 3, 0, 1)).reshape(HW, N, C)

    w1f = w1.astype(jnp.float32)
    w2f = w2.astype(jnp.float32)
    b1r = b1.reshape(1, Cse).astype(jnp.float32)
    b2r = b2.reshape(1, C).astype(jnp.float32)

    npar = 2 if N % 16 == 0 else 1
    Np = N // npar
    slab = Np * C * itemsize
    t1 = _largest_divisor_tile(HW, slab, 4 << 20)

    resident_need = HW * slab + 4 * t1 * slab + (4 << 20)
    if resident_need <= _VMEM_BUDGET:
        outt = _resident_forward(xt, w1f, b1r, w2f, b2r, npar, t1)
    else:
        outt = _two_pass_forward(xt, w1f, b1r, w2f, b2r, npar, t1)

    return jnp.transpose(outt.reshape(H, W, N, C), (2, 3, 0, 1))
